# baseline (speedup 1.0000x reference)
import math

import jax
import jax.numpy as jnp
from jax import lax
from jax.experimental import pallas as pl
from jax.experimental.pallas import tpu as pltpu
from jax.experimental.pallas import tpu_sc as plsc

N = 10000
E = 320000
F = 128
NC = 2
NS = 16
NW = NC * NS
EPT = E // NW
CH = 40
NCHUNK = 256
EPAD = NW * NCHUNK * CH
NHS = 10240

NAGG = N
RPT = 624
RTAIL = N - NS * RPT
BNC = 1.0 / math.sqrt(1.0 + 1e-5)

_MESH = dict(core_axis_name="c", subcore_axis_name="s")


def _sc_deg_body(dst_hbm, deg_out, dstbuf, degbuf):
    c = lax.axis_index("c")
    s = lax.axis_index("s")
    wid = s * NC + c
    zero16 = jnp.zeros((16,), jnp.float32)
    ones16 = jnp.ones((16,), jnp.float32)

    def zbody(i, carry):
        degbuf[pl.ds(i * 16, 16)] = zero16
        return carry

    lax.fori_loop(0, N // 16, zbody, 0)

    pltpu.sync_copy(dst_hbm.at[pl.ds(wid * EPT, EPT)], dstbuf)

    def body(i, carry):
        idx = dstbuf[pl.ds(i * 16, 16)]
        plsc.addupdate_scatter(degbuf, [idx], ones16)
        return carry

    lax.fori_loop(0, EPT // 16, body, 0)
    pltpu.sync_copy(degbuf, deg_out.at[wid])


def _sc_deg(dst):
    call = pl.kernel(
        _sc_deg_body,
        out_type=jax.ShapeDtypeStruct((NW, N), jnp.float32),
        mesh=plsc.VectorSubcoreMesh(**_MESH),
        scratch_types=[
            pltpu.VMEM((EPT,), jnp.int32),
            pltpu.VMEM((N,), jnp.float32),
        ],
        compiler_params=pltpu.CompilerParams(needs_layout_passes=False),
    )
    return call(dst)


NBUF = 4
SLAB = 32
NSLAB = NCHUNK // SLAB


def _sc_agg_body(hs_hbm, src_hbm, dst_hbm, zeros_hbm, agg_out,
                 src0, src1, dst0, dst1, agg_sh, rows0, rows1, rows2, rows3,
                 sem0, sem1, sem2, sem3, isem0, isem1, ssem0, ssem1,
                 ssem2, ssem3):
    rows = (rows0, rows1, rows2, rows3)
    sems = (sem0, sem1, sem2, sem3)
    ssems = (ssem0, ssem1, ssem2, ssem3)
    srcs = (src0, src1)
    dsts = (dst0, dst1)
    isems = (isem0, isem1)
    c = lax.axis_index("c")
    s = lax.axis_index("s")
    wid = s * NC + c

    pltpu.sync_copy(zeros_hbm.at[pl.ds(s * RPT, RPT)],
                    agg_sh.at[pl.ds(s * RPT, RPT)])

    @pl.when(s == 0)
    def _zero_tail():
        pltpu.sync_copy(zeros_hbm.at[pl.ds(NS * RPT, RTAIL + (NAGG - N))],
                        agg_sh.at[pl.ds(NS * RPT, RTAIL + (NAGG - N))])

    def fetch_slab(t, p):
        pltpu.async_copy(src_hbm.at[wid, t], srcs[p], isems[p])
        pltpu.async_copy(dst_hbm.at[wid, t], dsts[p], isems[p])

    def wait_slab(t, p):
        pltpu.make_async_copy(src_hbm.at[wid, t], srcs[p], isems[p]).wait()
        pltpu.make_async_copy(dst_hbm.at[wid, t], dsts[p], isems[p]).wait()

    def gather(q, j, b):
        pltpu.async_copy(hs_hbm.at[srcs[q].at[j]], rows[b], sems[b])

    def wait_gather(q, j, b):
        pltpu.make_async_copy(hs_hbm.at[srcs[q].at[j]], rows[b],
                              sems[b]).wait()

    def scatter(q, j, b):
        pltpu.async_copy(rows[b], agg_sh.at[dsts[q].at[j]], ssems[b],
                         add=True)

    def wait_scatter(q, j, b):
        pltpu.make_async_copy(rows[b], agg_sh.at[dsts[q].at[j]],
                              ssems[b]).wait()

    fetch_slab(0, 0)
    plsc.subcore_barrier()
    wait_slab(0, 0)
    for b in range(NBUF):
        gather(0, b, b)

    for t in range(NSLAB):
        p = t % 2
        pn = 1 - p
        if t + 1 < NSLAB:
            fetch_slab(t + 1, pn)

        def body(g, carry, p=p):
            j0 = NBUF * g
            for b in range(NBUF):
                wait_gather(p, j0 + b, b)
                scatter(p, j0 + b, b)
            for b in range(NBUF):
                wait_scatter(p, j0 + b, b)
                gather(p, j0 + b + NBUF, b)
            return carry

        lax.fori_loop(0, SLAB // NBUF - 1, body, 0)
        j0 = SLAB - NBUF
        if t + 1 < NSLAB:
            wait_slab(t + 1, pn)
            for b in range(NBUF):
                wait_gather(p, j0 + b, b)
                scatter(p, j0 + b, b)
            for b in range(NBUF):
                wait_scatter(p, j0 + b, b)
                gather(pn, b, b)
        else:
            for b in range(NBUF):
                wait_gather(p, j0 + b, b)
                scatter(p, j0 + b, b)
            for b in range(NBUF):
                wait_scatter(p, j0 + b, b)

    plsc.subcore_barrier()
    pltpu.sync_copy(agg_sh.at[pl.ds(s * RPT, RPT)],
                    agg_out.at[c, pl.ds(s * RPT, RPT)])

    @pl.when(s == 0)
    def _export_tail():
        pltpu.sync_copy(agg_sh.at[pl.ds(NS * RPT, RTAIL)],
                        agg_out.at[c, pl.ds(NS * RPT, RTAIL)])


def _sc_agg(hs, src4, dst4, zeros):
    call = pl.kernel(
        _sc_agg_body,
        out_type=jax.ShapeDtypeStruct((NC, N, F), jnp.float32),
        mesh=plsc.VectorSubcoreMesh(**_MESH),
        scratch_types=[
            pltpu.VMEM((SLAB, CH), jnp.int32),
            pltpu.VMEM((SLAB, CH), jnp.int32),
            pltpu.VMEM((SLAB, CH), jnp.int32),
            pltpu.VMEM((SLAB, CH), jnp.int32),
            pltpu.VMEM_SHARED((NAGG, F), jnp.float32),
            pltpu.VMEM((CH, F), jnp.float32),
            pltpu.VMEM((CH, F), jnp.float32),
            pltpu.VMEM((CH, F), jnp.float32),
            pltpu.VMEM((CH, F), jnp.float32),
            pltpu.SemaphoreType.DMA,
            pltpu.SemaphoreType.DMA,
            pltpu.SemaphoreType.DMA,
            pltpu.SemaphoreType.DMA,
            pltpu.SemaphoreType.DMA,
            pltpu.SemaphoreType.DMA,
            pltpu.SemaphoreType.DMA,
            pltpu.SemaphoreType.DMA,
            pltpu.SemaphoreType.DMA,
            pltpu.SemaphoreType.DMA,
        ],
    )
    return call(hs, src4, dst4, zeros)


def _tc0_body(degp_ref, x_ref, w_ref, h_ref, hs_ref, dinv_ref):
    deg = jnp.sum(degp_ref[...], axis=0) + 1.0
    dinv = lax.rsqrt(deg)[:, None]
    h = jnp.dot(x_ref[...], w_ref[...], preferred_element_type=jnp.float32)
    h_ref[...] = h
    hs_ref[pl.ds(0, N), :] = h * dinv
    hs_ref[pl.ds(N, NHS - N), :] = jnp.zeros((NHS - N, F), jnp.float32)
    dinv_ref[...] = dinv


def _tc0(degp, x, w):
    return pl.pallas_call(
        _tc0_body,
        out_shape=(
            jax.ShapeDtypeStruct((N, F), jnp.float32),
            jax.ShapeDtypeStruct((NHS, F), jnp.float32),
            jax.ShapeDtypeStruct((N, 1), jnp.float32),
        ),
    )(degp, x, w)


def _tc_mid_body(p_ref, h_ref, dinv_ref, b_ref, g_ref, bt_ref, w_ref,
                 hn_ref, hsn_ref):
    dinv = dinv_ref[...]
    agg = dinv * (p_ref[0] + p_ref[1]) + (dinv * dinv) * h_ref[...] + b_ref[...]
    xn = jnp.maximum(agg * (BNC * g_ref[...]) + bt_ref[...], 0.0)
    h = jnp.dot(xn, w_ref[...], preferred_element_type=jnp.float32)
    hn_ref[...] = h
    hsn_ref[pl.ds(0, N), :] = h * dinv
    hsn_ref[pl.ds(N, NHS - N), :] = jnp.zeros((NHS - N, F), jnp.float32)


def _tc_mid(p, h, dinv, b, g, bt, w):
    return pl.pallas_call(
        _tc_mid_body,
        out_shape=(
            jax.ShapeDtypeStruct((N, F), jnp.float32),
            jax.ShapeDtypeStruct((NHS, F), jnp.float32),
        ),
    )(p, h, dinv, b, g, bt, w)


def _tc_fin_body(p_ref, h_ref, dinv_ref, b_ref, g_ref, bt_ref,
                 we_ref, be_ref, wc1_ref, bc1_ref, wc2_ref, bc2_ref, out_ref):
    dinv = dinv_ref[...]
    agg = dinv * (p_ref[0] + p_ref[1]) + (dinv * dinv) * h_ref[...] + b_ref[...]
    xn = jnp.maximum(agg * (BNC * g_ref[...]) + bt_ref[...], 0.0)
    xg = jnp.sum(xn, axis=0, keepdims=True) * (1.0 / N)
    emb = jnp.maximum(
        jnp.dot(xg, we_ref[...], preferred_element_type=jnp.float32)
        + be_ref[...], 0.0)
    a = jnp.maximum(
        jnp.dot(emb, wc1_ref[...], preferred_element_type=jnp.float32)
        + bc1_ref[...], 0.0)
    o = jnp.sum(a[0] * wc2_ref[...][:, 0]) + bc2_ref[...][0]
    out_ref[...] = jnp.reshape(o, (1, 1))


def _tc_fin(p, h, dinv, b, g, bt, we, be, wc1, bc1, wc2, bc2):
    return pl.pallas_call(
        _tc_fin_body,
        out_shape=jax.ShapeDtypeStruct((1, 1), jnp.float32),
    )(p, h, dinv, b, g, bt, we, be, wc1, bc1, wc2, bc2)


def kernel(x, edge_index, batch, W1, b1, g1, bt1, W2, b2, g2, bt2,
           W3, b3, g3, bt3, We, be, Wc1, bc1, Wc2, bc2):
    src = edge_index[0]
    dst = edge_index[1]
    npad_t = (EPAD - E) // NW
    spad = (N + jnp.arange(NW * npad_t, dtype=jnp.int32) % (NHS - N)
            ).reshape(NW, npad_t)
    dpad = ((jnp.arange(NW * npad_t, dtype=jnp.int32) * 1237) % N
            ).reshape(NW, npad_t)
    src4 = jnp.concatenate([src.reshape(NW, EPT), spad],
                           axis=1).reshape(NW, NSLAB, SLAB, CH)
    dst4 = jnp.concatenate([dst.reshape(NW, EPT), dpad],
                           axis=1).reshape(NW, NSLAB, SLAB, CH)
    zeros = jnp.zeros((NAGG, F), jnp.float32)

    degp = _sc_deg(dst)
    h1, hs1, dinv = _tc0(degp, x, W1)
    p1 = _sc_agg(hs1, src4, dst4, zeros)
    h2, hs2 = _tc_mid(p1, h1, dinv, b1, g1, bt1, W2)
    p2 = _sc_agg(hs2, src4, dst4, zeros)
    h3, hs3 = _tc_mid(p2, h2, dinv, b2, g2, bt2, W3)
    p3 = _sc_agg(hs3, src4, dst4, zeros)
    return _tc_fin(p3, h3, dinv, b3, g3, bt3, We, be, Wc1, bc1, Wc2, bc2)


# speedup vs baseline: 1.1302x; 1.1302x over previous
import math

import jax
import jax.numpy as jnp
from jax import lax
from jax.experimental import pallas as pl
from jax.experimental.pallas import tpu as pltpu
from jax.experimental.pallas import tpu_sc as plsc

N = 10000
E = 320000
F = 128
NC = 2
NS = 16
NW = NC * NS
EPT = E // NW
CH = 40
NCHUNK = 256
EPAD = NW * NCHUNK * CH
NHS = 10240

NAGG = N
RPT = 624
RTAIL = N - NS * RPT
BNC = 1.0 / math.sqrt(1.0 + 1e-5)

_MESH = dict(core_axis_name="c", subcore_axis_name="s")


def _sc_deg_body(dst_hbm, deg_out, dstbuf, degbuf):
    c = lax.axis_index("c")
    s = lax.axis_index("s")
    wid = s * NC + c
    zero16 = jnp.zeros((16,), jnp.float32)
    ones16 = jnp.ones((16,), jnp.float32)

    def zbody(i, carry):
        degbuf[pl.ds(i * 16, 16)] = zero16
        return carry

    lax.fori_loop(0, N // 16, zbody, 0)

    pltpu.sync_copy(dst_hbm.at[pl.ds(wid * EPT, EPT)], dstbuf)

    def body(i, carry):
        idx = dstbuf[pl.ds(i * 16, 16)]
        plsc.addupdate_scatter(degbuf, [idx], ones16)
        return carry

    lax.fori_loop(0, EPT // 16, body, 0)
    pltpu.sync_copy(degbuf, deg_out.at[wid])


def _sc_deg(dst):
    call = pl.kernel(
        _sc_deg_body,
        out_type=jax.ShapeDtypeStruct((NW, N), jnp.float32),
        mesh=plsc.VectorSubcoreMesh(**_MESH),
        scratch_types=[
            pltpu.VMEM((EPT,), jnp.int32),
            pltpu.VMEM((N,), jnp.float32),
        ],
        compiler_params=pltpu.CompilerParams(needs_layout_passes=False),
    )
    return call(dst)


NBUF = 4
SLAB = 32
NSLAB = NCHUNK // SLAB


def _sc_agg_body(hs_hbm, src_hbm, dst_hbm, zeros_hbm, agg_out,
                 src0, src1, dst0, dst1, agg_sh, rows0, rows1, rows2, rows3,
                 sem0, sem1, sem2, sem3, isem0, isem1):
    rows = (rows0, rows1, rows2, rows3)
    sems = (sem0, sem1, sem2, sem3)
    srcs = (src0, src1)
    dsts = (dst0, dst1)
    isems = (isem0, isem1)
    c = lax.axis_index("c")
    s = lax.axis_index("s")
    wid = s * NC + c

    pltpu.sync_copy(zeros_hbm.at[pl.ds(s * RPT, RPT)],
                    agg_sh.at[pl.ds(s * RPT, RPT)])

    @pl.when(s == 0)
    def _zero_tail():
        pltpu.sync_copy(zeros_hbm.at[pl.ds(NS * RPT, RTAIL + (NAGG - N))],
                        agg_sh.at[pl.ds(NS * RPT, RTAIL + (NAGG - N))])

    def fetch_slab(t, p):
        pltpu.async_copy(src_hbm.at[wid, t], srcs[p], isems[p])
        pltpu.async_copy(dst_hbm.at[wid, t], dsts[p], isems[p])

    def wait_slab(t, p):
        pltpu.make_async_copy(src_hbm.at[wid, t], srcs[p], isems[p]).wait()
        pltpu.make_async_copy(dst_hbm.at[wid, t], dsts[p], isems[p]).wait()

    def gather(q, j, b):
        pltpu.async_copy(hs_hbm.at[srcs[q].at[j]], rows[b], sems[b])

    def wait_gather(q, j, b):
        pltpu.make_async_copy(hs_hbm.at[srcs[q].at[j]], rows[b],
                              sems[b]).wait()

    def scatter(q, j, b):
        pltpu.sync_copy(rows[b], agg_sh.at[dsts[q].at[j]], add=True)

    fetch_slab(0, 0)
    plsc.subcore_barrier()
    wait_slab(0, 0)
    for b in range(NBUF):
        gather(0, b, b)

    for t in range(NSLAB):
        p = t % 2
        pn = 1 - p
        if t + 1 < NSLAB:
            fetch_slab(t + 1, pn)

        def body(g, carry, p=p):
            j0 = NBUF * g
            for b in range(NBUF):
                wait_gather(p, j0 + b, b)
                scatter(p, j0 + b, b)
                gather(p, j0 + b + NBUF, b)
            return carry

        lax.fori_loop(0, SLAB // NBUF - 1, body, 0)
        j0 = SLAB - NBUF
        if t + 1 < NSLAB:
            wait_slab(t + 1, pn)
            for b in range(NBUF):
                wait_gather(p, j0 + b, b)
                scatter(p, j0 + b, b)
                gather(pn, b, b)
        else:
            for b in range(NBUF):
                wait_gather(p, j0 + b, b)
                scatter(p, j0 + b, b)

    plsc.subcore_barrier()
    pltpu.sync_copy(agg_sh.at[pl.ds(s * RPT, RPT)],
                    agg_out.at[c, pl.ds(s * RPT, RPT)])

    @pl.when(s == 0)
    def _export_tail():
        pltpu.sync_copy(agg_sh.at[pl.ds(NS * RPT, RTAIL)],
                        agg_out.at[c, pl.ds(NS * RPT, RTAIL)])


def _sc_agg(hs, src4, dst4, zeros):
    call = pl.kernel(
        _sc_agg_body,
        out_type=jax.ShapeDtypeStruct((NC, N, F), jnp.float32),
        mesh=plsc.VectorSubcoreMesh(**_MESH),
        scratch_types=[
            pltpu.VMEM((SLAB, CH), jnp.int32),
            pltpu.VMEM((SLAB, CH), jnp.int32),
            pltpu.VMEM((SLAB, CH), jnp.int32),
            pltpu.VMEM((SLAB, CH), jnp.int32),
            pltpu.VMEM_SHARED((NAGG, F), jnp.float32),
            pltpu.VMEM((CH, F), jnp.float32),
            pltpu.VMEM((CH, F), jnp.float32),
            pltpu.VMEM((CH, F), jnp.float32),
            pltpu.VMEM((CH, F), jnp.float32),
            pltpu.SemaphoreType.DMA,
            pltpu.SemaphoreType.DMA,
            pltpu.SemaphoreType.DMA,
            pltpu.SemaphoreType.DMA,
            pltpu.SemaphoreType.DMA,
            pltpu.SemaphoreType.DMA,
        ],
    )
    return call(hs, src4, dst4, zeros)


def _tc0_body(degp_ref, x_ref, w_ref, h_ref, hs_ref, dinv_ref):
    deg = jnp.sum(degp_ref[...], axis=0) + 1.0
    dinv = lax.rsqrt(deg)[:, None]
    h = jnp.dot(x_ref[...], w_ref[...], preferred_element_type=jnp.float32)
    h_ref[...] = h
    hs_ref[pl.ds(0, N), :] = h * dinv
    hs_ref[pl.ds(N, NHS - N), :] = jnp.zeros((NHS - N, F), jnp.float32)
    dinv_ref[...] = dinv


def _tc0(degp, x, w):
    return pl.pallas_call(
        _tc0_body,
        out_shape=(
            jax.ShapeDtypeStruct((N, F), jnp.float32),
            jax.ShapeDtypeStruct((NHS, F), jnp.float32),
            jax.ShapeDtypeStruct((N, 1), jnp.float32),
        ),
    )(degp, x, w)


def _tc_mid_body(p_ref, h_ref, dinv_ref, b_ref, g_ref, bt_ref, w_ref,
                 hn_ref, hsn_ref):
    dinv = dinv_ref[...]
    agg = dinv * (p_ref[0] + p_ref[1]) + (dinv * dinv) * h_ref[...] + b_ref[...]
    xn = jnp.maximum(agg * (BNC * g_ref[...]) + bt_ref[...], 0.0)
    h = jnp.dot(xn, w_ref[...], preferred_element_type=jnp.float32)
    hn_ref[...] = h
    hsn_ref[pl.ds(0, N), :] = h * dinv
    hsn_ref[pl.ds(N, NHS - N), :] = jnp.zeros((NHS - N, F), jnp.float32)


def _tc_mid(p, h, dinv, b, g, bt, w):
    return pl.pallas_call(
        _tc_mid_body,
        out_shape=(
            jax.ShapeDtypeStruct((N, F), jnp.float32),
            jax.ShapeDtypeStruct((NHS, F), jnp.float32),
        ),
    )(p, h, dinv, b, g, bt, w)


def _tc_fin_body(p_ref, h_ref, dinv_ref, b_ref, g_ref, bt_ref,
                 we_ref, be_ref, wc1_ref, bc1_ref, wc2_ref, bc2_ref, out_ref):
    dinv = dinv_ref[...]
    agg = dinv * (p_ref[0] + p_ref[1]) + (dinv * dinv) * h_ref[...] + b_ref[...]
    xn = jnp.maximum(agg * (BNC * g_ref[...]) + bt_ref[...], 0.0)
    xg = jnp.sum(xn, axis=0, keepdims=True) * (1.0 / N)
    emb = jnp.maximum(
        jnp.dot(xg, we_ref[...], preferred_element_type=jnp.float32)
        + be_ref[...], 0.0)
    a = jnp.maximum(
        jnp.dot(emb, wc1_ref[...], preferred_element_type=jnp.float32)
        + bc1_ref[...], 0.0)
    o = jnp.sum(a[0] * wc2_ref[...][:, 0]) + bc2_ref[...][0]
    out_ref[...] = jnp.reshape(o, (1, 1))


def _tc_fin(p, h, dinv, b, g, bt, we, be, wc1, bc1, wc2, bc2):
    return pl.pallas_call(
        _tc_fin_body,
        out_shape=jax.ShapeDtypeStruct((1, 1), jnp.float32),
    )(p, h, dinv, b, g, bt, we, be, wc1, bc1, wc2, bc2)


def kernel(x, edge_index, batch, W1, b1, g1, bt1, W2, b2, g2, bt2,
           W3, b3, g3, bt3, We, be, Wc1, bc1, Wc2, bc2):
    src = edge_index[0]
    dst = edge_index[1]
    npad_t = (EPAD - E) // NW
    spad = (N + jnp.arange(NW * npad_t, dtype=jnp.int32) % (NHS - N)
            ).reshape(NW, npad_t)
    dpad = ((jnp.arange(NW * npad_t, dtype=jnp.int32) * 1237) % N
            ).reshape(NW, npad_t)
    src4 = jnp.concatenate([src.reshape(NW, EPT), spad],
                           axis=1).reshape(NW, NSLAB, SLAB, CH)
    dst4 = jnp.concatenate([dst.reshape(NW, EPT), dpad],
                           axis=1).reshape(NW, NSLAB, SLAB, CH)
    zeros = jnp.zeros((NAGG, F), jnp.float32)

    degp = _sc_deg(dst)
    h1, hs1, dinv = _tc0(degp, x, W1)
    p1 = _sc_agg(hs1, src4, dst4, zeros)
    h2, hs2 = _tc_mid(p1, h1, dinv, b1, g1, bt1, W2)
    p2 = _sc_agg(hs2, src4, dst4, zeros)
    h3, hs3 = _tc_mid(p2, h2, dinv, b2, g2, bt2, W3)
    p3 = _sc_agg(hs3, src4, dst4, zeros)
    return _tc_fin(p3, h3, dinv, b3, g3, bt3, We, be, Wc1, bc1, Wc2, bc2)
